# baseline (speedup 1.0000x reference)
import functools

import jax
import jax.numpy as jnp
from jax import lax
from jax.experimental import pallas as pl
from jax.experimental.pallas import tpu as pltpu
from jax.experimental.pallas import tpu_sc as plsc

_N = 200
_D = 64
_B = 4096
_NC = 2
_NS = 16
_NW = _NC * _NS
_BBLK = 128
_NBB = _B // _BBLK
_NGRP = 8
_GROUPS = (_N // _NGRP) * _NBB
_GPW = _GROUPS // _NW


def _transpose16(vs, lane, left_idx, right_idx, masks):
    for s in (1, 2, 4, 8):
        nv = list(vs)
        for r in range(16):
            if r & s:
                continue
            a, b = vs[r], vs[r + s]
            ta = b.at[left_idx[s]].get(mode="promise_in_bounds")
            tb = a.at[right_idx[s]].get(mode="promise_in_bounds")
            nv[r] = jnp.where(masks[s], a, ta)
            nv[r + s] = jnp.where(masks[s], tb, b)
        vs = nv
    return vs


def _emb_body(xT_hbm, item_hbm, pos_hbm, out_hbm, pos_v, idx0, idx1,
              rows0, rows1, obuf0, obuf1, isem, gsem0, gsem1, osem0, osem1):
    wid = lax.axis_index("s") * _NC + lax.axis_index("c")
    pltpu.sync_copy(pos_hbm, pos_v)
    g0 = wid * _GPW
    idx_bufs = (idx0, idx1)
    rows_bufs = (rows0, rows1)
    gsems = (gsem0, gsem1)
    obufs = (obuf0, obuf1)
    osems = (osem0, osem1)

    lane = lax.iota(jnp.int32, 16)
    left_idx = {s: (lane - s) & 15 for s in (1, 2, 4, 8)}
    right_idx = {s: (lane + s) & 15 for s in (1, 2, 4, 8)}
    masks = {s: (lane & s) == 0 for s in (1, 2, 4, 8)}

    def idx_src(g):
        gid = g0 + g
        ng = gid // _NBB
        b0 = (gid % _NBB) * _BBLK
        return xT_hbm.at[pl.ds(ng * _NGRP, _NGRP), pl.ds(b0, _BBLK)]

    pltpu.async_copy(idx_src(0), idx0, isem)

    def run_group(g, idx_v, prefetch_next):
        gid = g0 + g
        ng = gid // _NBB
        b0 = (gid % _NBB) * _BBLK
        pltpu.make_async_copy(idx_src(g), idx_v, isem).wait()
        prefetch_next()
        pltpu.async_copy(item_hbm.at[idx_v.at[0]], rows_bufs[0], gsems[0])
        for j in range(_NGRP):
            n = ng * _NGRP + j
            par = j % 2
            rows_v = rows_bufs[par]
            if j + 1 < _NGRP:
                pltpu.async_copy(
                    item_hbm.at[idx_v.at[j + 1]],
                    rows_bufs[1 - par],
                    gsems[1 - par],
                )
            pltpu.make_async_copy(
                item_hbm.at[idx_v.at[j]], rows_v, gsems[par]
            ).wait()
            hpar = (j // 2) % 2
            obuf_pair = obufs[hpar]
            if j % 2 == 0 and j >= 4:
                pltpu.make_async_copy(
                    obuf_pair,
                    out_hbm.at[pl.ds(n, 2), :, pl.ds(b0, _BBLK)],
                    osems[hpar],
                ).wait()
            ps = [pos_v[n, pl.ds(16 * k, 16)] for k in range(_D // 16)]

            def rb_body(rb, carry2):
                r0 = rb * 16
                for k in range(_D // 16):
                    vs = [
                        rows_v[r0 + i, pl.ds(16 * k, 16)] + ps[k]
                        for i in range(16)
                    ]
                    vs = _transpose16(vs, lane, left_idx, right_idx, masks)
                    for i in range(16):
                        obuf_pair[j % 2, 16 * k + i, pl.ds(r0, 16)] = vs[i]
                return carry2

            lax.fori_loop(0, _BBLK // 16, rb_body, 0)
            if j % 2 == 1:
                pltpu.async_copy(
                    obuf_pair,
                    out_hbm.at[pl.ds(n - 1, 2), :, pl.ds(b0, _BBLK)],
                    osems[hpar],
                )
        for h in (_NGRP // 2 - 2, _NGRP // 2 - 1):
            n0 = ng * _NGRP + 2 * h
            pltpu.make_async_copy(
                obufs[h % 2],
                out_hbm.at[pl.ds(n0, 2), :, pl.ds(b0, _BBLK)],
                osems[h % 2],
            ).wait()

    def pair_body(p, carry):
        for q in range(2):
            g = p * 2 + q

            def prefetch():
                @pl.when(g + 1 < _GPW)
                def _():
                    pltpu.async_copy(idx_src(g + 1), idx_bufs[1 - q], isem)

            run_group(g, idx_bufs[q], prefetch)
        return carry

    lax.fori_loop(0, _GPW // 2, pair_body, 0)
    if _GPW % 2:
        run_group(_GPW - 1, idx_bufs[0], lambda: None)


@jax.jit
def _emb_call(xT, item_table, pos_table):
    mesh = plsc.VectorSubcoreMesh(
        core_axis_name="c", subcore_axis_name="s"
    )
    run = pl.kernel(
        _emb_body,
        out_type=jax.ShapeDtypeStruct((_N, _D, _B), jnp.float32),
        mesh=mesh,
        compiler_params=pltpu.CompilerParams(needs_layout_passes=False),
        scratch_types=[
            pltpu.VMEM((_N, _D), jnp.float32),
            pltpu.VMEM((_NGRP, _BBLK), jnp.int32),
            pltpu.VMEM((_NGRP, _BBLK), jnp.int32),
            pltpu.VMEM((_BBLK, 2 * _D), jnp.float32),
            pltpu.VMEM((_BBLK, 2 * _D), jnp.float32),
            pltpu.VMEM((2, _D, _BBLK), jnp.float32),
            pltpu.VMEM((2, _D, _BBLK), jnp.float32),
            pltpu.SemaphoreType.DMA,
            pltpu.SemaphoreType.DMA,
            pltpu.SemaphoreType.DMA,
            pltpu.SemaphoreType.DMA,
            pltpu.SemaphoreType.DMA,
        ],
    )
    return run(xT, item_table, pos_table)


def kernel(x, item_table, pos_table):
    xT = jnp.transpose(x.astype(jnp.int32))
    item_pad = jnp.pad(item_table, ((0, 7), (0, _D)))
    out = _emb_call(xT, item_pad, pos_table)
    return jnp.transpose(out, (2, 0, 1))


# speedup vs baseline: 1.0097x; 1.0097x over previous
import functools

import jax
import jax.numpy as jnp
from jax import lax
from jax.experimental import pallas as pl
from jax.experimental.pallas import tpu as pltpu
from jax.experimental.pallas import tpu_sc as plsc

_N = 200
_D = 64
_B = 4096
_NC = 2
_NS = 16
_NW = _NC * _NS
_BBLK = 128
_NBB = _B // _BBLK
_NGRP = 8
_GROUPS = (_N // _NGRP) * _NBB
_GPW = _GROUPS // _NW


def _transpose16(vs, lane, left_idx, right_idx, masks):
    for s in (1, 2, 4, 8):
        nv = list(vs)
        for r in range(16):
            if r & s:
                continue
            a, b = vs[r], vs[r + s]
            ta = b.at[left_idx[s]].get(mode="promise_in_bounds")
            tb = a.at[right_idx[s]].get(mode="promise_in_bounds")
            nv[r] = jnp.where(masks[s], a, ta)
            nv[r + s] = jnp.where(masks[s], tb, b)
        vs = nv
    return vs


def _emb_body(xT_hbm, item_hbm, pos_hbm, out_hbm, pos_v, idx0, idx1,
              rows0, rows1, obuf0, obuf1, isem, gsem0, gsem1, osem0, osem1):
    wid = lax.axis_index("s") * _NC + lax.axis_index("c")
    pltpu.sync_copy(pos_hbm, pos_v)
    g0 = wid * _GPW
    idx_bufs = (idx0, idx1)
    rows_bufs = (rows0, rows1)
    gsems = (gsem0, gsem1)
    obufs = (obuf0, obuf1)
    osems = (osem0, osem1)

    lane = lax.iota(jnp.int32, 16)
    left_idx = {s: (lane - s) & 15 for s in (1, 2, 4, 8)}
    right_idx = {s: (lane + s) & 15 for s in (1, 2, 4, 8)}
    masks = {s: (lane & s) == 0 for s in (1, 2, 4, 8)}

    def idx_src(g):
        gid = g0 + g
        ng = gid // _NBB
        b0 = (gid % _NBB) * _BBLK
        return xT_hbm.at[pl.ds(ng * _NGRP, _NGRP), pl.ds(b0, _BBLK)]

    pltpu.async_copy(idx_src(0), idx0, isem)

    def run_group(g, idx_v, prefetch_next):
        gid = g0 + g
        ng = gid // _NBB
        b0 = (gid % _NBB) * _BBLK
        pltpu.make_async_copy(idx_src(g), idx_v, isem).wait()
        prefetch_next()
        pltpu.async_copy(item_hbm.at[idx_v.at[0]], rows_bufs[0], gsems[0])
        for j in range(_NGRP):
            n = ng * _NGRP + j
            par = j % 2
            rows_v = rows_bufs[par]
            if j + 1 < _NGRP:
                pltpu.async_copy(
                    item_hbm.at[idx_v.at[j + 1]],
                    rows_bufs[1 - par],
                    gsems[1 - par],
                )
            pltpu.make_async_copy(
                item_hbm.at[idx_v.at[j]], rows_v, gsems[par]
            ).wait()
            obuf_v = obufs[par]
            if j >= 2:
                pltpu.make_async_copy(
                    obuf_v, out_hbm.at[n, :, pl.ds(b0, _BBLK)], osems[par]
                ).wait()
            ps = [pos_v[n, pl.ds(16 * k, 16)] for k in range(_D // 16)]

            def rb_body(rb, carry2):
                r0 = rb * 16
                for k in range(_D // 16):
                    vs = [
                        rows_v[r0 + i, pl.ds(16 * k, 16)] + ps[k]
                        for i in range(16)
                    ]
                    vs = _transpose16(vs, lane, left_idx, right_idx, masks)
                    for i in range(16):
                        obuf_v[16 * k + i, pl.ds(r0, 16)] = vs[i]
                return carry2

            lax.fori_loop(0, _BBLK // 16, rb_body, 0)
            pltpu.async_copy(
                obuf_v, out_hbm.at[n, :, pl.ds(b0, _BBLK)], osems[par]
            )
        for j in (_NGRP - 2, _NGRP - 1):
            n = ng * _NGRP + j
            pltpu.make_async_copy(
                obufs[j % 2], out_hbm.at[n, :, pl.ds(b0, _BBLK)], osems[j % 2]
            ).wait()

    def pair_body(p, carry):
        for q in range(2):
            g = p * 2 + q

            def prefetch():
                @pl.when(g + 1 < _GPW)
                def _():
                    pltpu.async_copy(idx_src(g + 1), idx_bufs[1 - q], isem)

            run_group(g, idx_bufs[q], prefetch)
        return carry

    lax.fori_loop(0, _GPW // 2, pair_body, 0)
    if _GPW % 2:
        run_group(_GPW - 1, idx_bufs[0], lambda: None)


@jax.jit
def _emb_call(xT, item_table, pos_table):
    mesh = plsc.VectorSubcoreMesh(
        core_axis_name="c", subcore_axis_name="s"
    )
    run = pl.kernel(
        _emb_body,
        out_type=jax.ShapeDtypeStruct((_N, _D, _B), jnp.float32),
        mesh=mesh,
        compiler_params=pltpu.CompilerParams(needs_layout_passes=False),
        scratch_types=[
            pltpu.VMEM((_N, _D), jnp.float32),
            pltpu.VMEM((_NGRP, _BBLK), jnp.int32),
            pltpu.VMEM((_NGRP, _BBLK), jnp.int32),
            pltpu.VMEM((_BBLK, 2 * _D), jnp.float32),
            pltpu.VMEM((_BBLK, 2 * _D), jnp.float32),
            pltpu.VMEM((_D, _BBLK), jnp.float32),
            pltpu.VMEM((_D, _BBLK), jnp.float32),
            pltpu.SemaphoreType.DMA,
            pltpu.SemaphoreType.DMA,
            pltpu.SemaphoreType.DMA,
            pltpu.SemaphoreType.DMA,
            pltpu.SemaphoreType.DMA,
        ],
    )
    return run(xT, item_table, pos_table)


def kernel(x, item_table, pos_table):
    xT = jnp.transpose(x.astype(jnp.int32))
    item_pad = jnp.pad(item_table, ((0, 7), (0, _D)))
    out = _emb_call(xT, item_pad, pos_table)
    return jnp.transpose(out, (2, 0, 1))
